# baseline (speedup 1.0000x reference)
import functools

import jax
import jax.numpy as jnp
from jax import lax
from jax.experimental import pallas as pl
from jax.experimental.pallas import tpu as pltpu
from jax.experimental.pallas import tpu_sc as plsc

N = 10000
D = 128
L = 3
NPAD = 10112
EPG = 64
NC = 2
NS = 16
NW = NC * NS
E = 320000
G = 160
EPW = G * EPG
EPAD = NW * EPW
RPT = NPAD // NS


def _mp_body(h_hbm, idx_hbm, ctab_hbm, out_hbm,
             i30, i31, dv0, dv1, hb0, hb1, eb0, eb1, mb0, mb1, agg,
             ix0, ix1, gh0, ge0, gh1, ge1, sc0, sc1):
    cid = lax.axis_index("c")
    sid = lax.axis_index("s")
    wid = sid * NC + cid

    def zbody(r, _):
        for c in range(D // 16):
            eb0[r, pl.ds(c * 16, 16)] = jnp.zeros((16,), jnp.float32)
        return 0
    lax.fori_loop(0, EPG, zbody, 0)
    for k in range(RPT // EPG):
        pltpu.sync_copy(eb0, agg.at[pl.ds(sid * RPT + k * EPG, EPG)])
    rem = RPT - (RPT // EPG) * EPG
    if rem:
        pltpu.sync_copy(eb0.at[pl.ds(0, rem)],
                        agg.at[pl.ds(sid * RPT + (RPT // EPG) * EPG, rem)])
    plsc.subcore_barrier()

    bufs = ((i30, dv0, hb0, eb0, mb0, ix0, gh0, ge0, sc0),
            (i31, dv1, hb1, eb1, mb1, ix1, gh1, ge1, sc1))

    for b in range(2):
        i3, dv, hb, eb, mb, ix, gh, ge, sc = bufs[b]
        pltpu.async_copy(idx_hbm.at[wid, b], i3, ix)
    for b in range(2):
        i3, dv, hb, eb, mb, ix, gh, ge, sc = bufs[b]
        pltpu.make_async_copy(idx_hbm.at[wid, b], i3, ix).wait()
        pltpu.async_copy(h_hbm.at[i3.at[pl.ds(0, EPG)]], hb, gh)
        pltpu.async_copy(ctab_hbm.at[i3.at[pl.ds(EPG, EPG)]], eb, ge)

    def pbody(p, _):
        for b in range(2):
            i3, dv, hb, eb, mb, ix, gh, ge, sc = bufs[b]
            g = 2 * p + b
            pltpu.make_async_copy(h_hbm.at[i3.at[pl.ds(0, EPG)]], hb, gh).wait()
            pltpu.make_async_copy(ctab_hbm.at[i3.at[pl.ds(EPG, EPG)]], eb, ge).wait()

            @pl.when(p > 0)
            def _():
                pltpu.make_async_copy(mb, agg.at[dv], sc).wait()

            for c in range(EPG // 16):
                dv[pl.ds(c * 16, 16)] = i3[pl.ds(2 * EPG + c * 16, 16)]

            @pl.when(g + 2 < G)
            def _():
                pltpu.async_copy(idx_hbm.at[wid, g + 2], i3, ix)

            def cbody(r2, _):
                r = 2 * r2
                for rr in range(2):
                    for c in range(D // 16):
                        s = pl.ds(c * 16, 16)
                        mb[r + rr, s] = jnp.maximum(hb[r + rr, s] + eb[r + rr, s], 0.0)
                return 0
            lax.fori_loop(0, EPG // 2, cbody, 0)

            pltpu.async_copy(mb, agg.at[dv], sc, add=True)

            @pl.when(g + 2 < G)
            def _():
                pltpu.make_async_copy(idx_hbm.at[wid, g + 2], i3, ix).wait()
                pltpu.async_copy(h_hbm.at[i3.at[pl.ds(0, EPG)]], hb, gh)
                pltpu.async_copy(ctab_hbm.at[i3.at[pl.ds(EPG, EPG)]], eb, ge)
        return 0
    lax.fori_loop(0, G // 2, pbody, 0)

    pltpu.make_async_copy(mb0, agg.at[dv0], sc0).wait()
    pltpu.make_async_copy(mb1, agg.at[dv1], sc1).wait()

    plsc.subcore_barrier()
    pltpu.sync_copy(agg.at[pl.ds(sid * RPT, RPT)],
                    out_hbm.at[cid, pl.ds(sid * RPT, RPT)])


def _mp_call(h, idxp, ctab_l):
    mesh = plsc.VectorSubcoreMesh(core_axis_name="c", subcore_axis_name="s")
    f = pl.kernel(
        _mp_body,
        out_type=jax.ShapeDtypeStruct((NC, NPAD, D), jnp.float32),
        mesh=mesh,
        scratch_types=[
            pltpu.VMEM((3 * EPG,), jnp.int32),
            pltpu.VMEM((3 * EPG,), jnp.int32),
            pltpu.VMEM((EPG,), jnp.int32),
            pltpu.VMEM((EPG,), jnp.int32),
            pltpu.VMEM((EPG, D), jnp.float32),
            pltpu.VMEM((EPG, D), jnp.float32),
            pltpu.VMEM((EPG, D), jnp.float32),
            pltpu.VMEM((EPG, D), jnp.float32),
            pltpu.VMEM((EPG, D), jnp.float32),
            pltpu.VMEM((EPG, D), jnp.float32),
            pltpu.VMEM_SHARED((NPAD, D), jnp.float32),
            pltpu.SemaphoreType.DMA,
            pltpu.SemaphoreType.DMA,
            pltpu.SemaphoreType.DMA,
            pltpu.SemaphoreType.DMA,
            pltpu.SemaphoreType.DMA,
            pltpu.SemaphoreType.DMA,
            pltpu.SemaphoreType.DMA,
            pltpu.SemaphoreType.DMA,
        ],
    )
    return f(h, idxp, ctab_l)


def _mlp_body(relu_out, h_ref, a_ref, w1_ref, b1_ref, g1_ref, t1_ref,
              w2_ref, b2_ref, go_ref, to_ref, eps_ref, out_ref):
    h = h_ref[...]
    agg = a_ref[0, 0:N, :] + a_ref[1, 0:N, :]
    z0 = (1.0 + eps_ref[0, 0]) * h + agg
    z1 = jnp.dot(z0, w1_ref[...], preferred_element_type=jnp.float32) + b1_ref[...]
    mu = jnp.mean(z1, axis=0, keepdims=True)
    var = jnp.mean((z1 - mu) ** 2, axis=0, keepdims=True)
    z1 = (z1 - mu) / jnp.sqrt(var + 1e-5) * g1_ref[...] + t1_ref[...]
    z1 = jnp.maximum(z1, 0.0)
    z2 = jnp.dot(z1, w2_ref[...], preferred_element_type=jnp.float32) + b2_ref[...]
    mu2 = jnp.mean(z2, axis=0, keepdims=True)
    var2 = jnp.mean((z2 - mu2) ** 2, axis=0, keepdims=True)
    z2 = (z2 - mu2) / jnp.sqrt(var2 + 1e-5) * go_ref[...] + to_ref[...]
    if relu_out:
        z2 = jnp.maximum(z2, 0.0)
    out_ref[...] = z2 + h


def _mlp_call(h, parts, w1, b1v, g1v, t1v, w2, b2v, gov, tov, eps_l, relu_out):
    body = functools.partial(_mlp_body, relu_out)
    vspec = pl.BlockSpec(memory_space=pltpu.VMEM)
    return pl.pallas_call(
        body,
        out_shape=jax.ShapeDtypeStruct((N, D), jnp.float32),
        in_specs=[vspec] * 10 + [pl.BlockSpec(memory_space=pltpu.SMEM)],
        out_specs=vspec,
    )(h, parts, w1, b1v, g1v, t1v, w2, b2v, gov, tov, eps_l)


def kernel(x, edge_index, edge_attr, batch, eps, W1, b1, g1, bt1, W2, b2, bond_emb, g_out, bt_out):
    src = edge_index[0]
    dst = edge_index[1]
    cidx = edge_attr[:, 0] * 25 + edge_attr[:, 1] * 5 + edge_attr[:, 2]
    srcp = jnp.pad(src, (0, EPAD - E)).reshape(NW, G, 1, EPG)
    cidxp = jnp.pad(cidx, (0, EPAD - E)).reshape(NW, G, 1, EPG)
    dstp = jnp.pad(dst, (0, EPAD - E), constant_values=N).reshape(NW, G, 1, EPG)
    idxp = jnp.concatenate([srcp, cidxp, dstp], axis=2).reshape(NW, G, 3 * EPG)
    ctab = (bond_emb[:, 0][:, :, None, None, :]
            + bond_emb[:, 1][:, None, :, None, :]
            + bond_emb[:, 2][:, None, None, :, :]).reshape(L, 125, D)
    ctab = jnp.pad(ctab, ((0, 0), (0, 3), (0, 0)))

    h = x
    for l in range(L):
        parts = _mp_call(h, idxp, ctab[l])
        h = _mlp_call(h, parts,
                      W1[l], b1[l][None], g1[l][None], bt1[l][None],
                      W2[l], b2[l][None], g_out[l][None], bt_out[l][None],
                      eps[l].reshape(1, 1), relu_out=(l < L - 1))
    return h


# speedup vs baseline: 1.0649x; 1.0649x over previous
import functools

import jax
import jax.numpy as jnp
from jax import lax
from jax.experimental import pallas as pl
from jax.experimental.pallas import tpu as pltpu
from jax.experimental.pallas import tpu_sc as plsc

N = 10000
D = 128
L = 3
NPAD = 10008
EPG = 128
NC = 2
NS = 16
NW = NC * NS
E = 320000
G = 80
EPW = G * EPG
EPAD = NW * EPW
RPT = 632


def _mp_body(h_hbm, idx_hbm, ctab_hbm, out_hbm,
             ib0, ib1, dv, hb0, hb1, eb, agg,
             ix0, ix1, gh0, gh1, ge):
    cid = lax.axis_index("c")
    sid = lax.axis_index("s")
    wid = sid * NC + cid
    base = jnp.minimum(sid * RPT, NPAD - RPT)

    def zbody(r, _):
        for c in range(D // 16):
            eb[r, pl.ds(c * 16, 16)] = jnp.zeros((16,), jnp.float32)
        return 0
    lax.fori_loop(0, EPG, zbody, 0)
    for k in range(RPT // EPG):
        pltpu.sync_copy(eb, agg.at[pl.ds(base + k * EPG, EPG)])
    rem = RPT - (RPT // EPG) * EPG
    if rem:
        pltpu.sync_copy(eb.at[pl.ds(0, rem)],
                        agg.at[pl.ds(base + (RPT // EPG) * EPG, rem)])
    plsc.subcore_barrier()

    ibs = (ib0, ib1)
    hbs = (hb0, hb1)
    ixs = (ix0, ix1)
    ghs = (gh0, gh1)

    pltpu.async_copy(idx_hbm.at[wid, 0], ib0, ix0)
    pltpu.async_copy(idx_hbm.at[wid, 1], ib1, ix1)
    pltpu.make_async_copy(idx_hbm.at[wid, 0], ib0, ix0).wait()
    pltpu.async_copy(h_hbm.at[ib0.at[pl.ds(0, EPG)]], hb0, gh0)
    pltpu.async_copy(ctab_hbm.at[ib0.at[pl.ds(EPG, EPG)]], eb, ge)

    def pbody(p, _):
        for b in range(2):
            g = 2 * p + b
            if True:
                ib, hb, ix, gh = ibs[b], hbs[b], ixs[b], ghs[b]
                ibn, hbn, ixn, ghn = ibs[1 - b], hbs[1 - b], ixs[1 - b], ghs[1 - b]

                @pl.when(g + 1 < G)
                def _():
                    pltpu.make_async_copy(idx_hbm.at[wid, g + 1], ibn, ixn).wait()
                    pltpu.async_copy(h_hbm.at[ibn.at[pl.ds(0, EPG)]], hbn, ghn)

                pltpu.make_async_copy(h_hbm.at[ib.at[pl.ds(0, EPG)]], hb, gh).wait()
                pltpu.make_async_copy(ctab_hbm.at[ib.at[pl.ds(EPG, EPG)]], eb, ge).wait()

                for c in range(EPG // 16):
                    dv[pl.ds(c * 16, 16)] = ib[pl.ds(2 * EPG + c * 16, 16)]

                @pl.when(g + 2 < G)
                def _():
                    pltpu.async_copy(idx_hbm.at[wid, g + 2], ib, ix)

                def cbody(r2, _):
                    r = 2 * r2
                    for rr in range(2):
                        for c in range(D // 16):
                            s = pl.ds(c * 16, 16)
                            hb[r + rr, s] = jnp.maximum(hb[r + rr, s] + eb[r + rr, s], 0.0)
                    return 0
                lax.fori_loop(0, EPG // 2, cbody, 0)

                @pl.when(g + 1 < G)
                def _():
                    pltpu.async_copy(ctab_hbm.at[ibn.at[pl.ds(EPG, EPG)]], eb, ge)

                pltpu.sync_copy(hb, agg.at[dv], add=True)
        return 0
    lax.fori_loop(0, G // 2, pbody, 0)

    plsc.subcore_barrier()
    pltpu.sync_copy(agg.at[pl.ds(base, RPT)],
                    out_hbm.at[cid, pl.ds(base, RPT)])


def _mp_call(h, idxp, ctab_l):
    mesh = plsc.VectorSubcoreMesh(core_axis_name="c", subcore_axis_name="s")
    f = pl.kernel(
        _mp_body,
        out_type=jax.ShapeDtypeStruct((NC, NPAD, D), jnp.float32),
        mesh=mesh,
        scratch_types=[
            pltpu.VMEM((3 * EPG,), jnp.int32),
            pltpu.VMEM((3 * EPG,), jnp.int32),
            pltpu.VMEM((EPG,), jnp.int32),
            pltpu.VMEM((EPG, D), jnp.float32),
            pltpu.VMEM((EPG, D), jnp.float32),
            pltpu.VMEM((EPG, D), jnp.float32),
            pltpu.VMEM_SHARED((NPAD, D), jnp.float32),
            pltpu.SemaphoreType.DMA,
            pltpu.SemaphoreType.DMA,
            pltpu.SemaphoreType.DMA,
            pltpu.SemaphoreType.DMA,
            pltpu.SemaphoreType.DMA,
        ],
    )
    return f(h, idxp, ctab_l)


def _mlp_body(relu_out, h_ref, a_ref, w1_ref, b1_ref, g1_ref, t1_ref,
              w2_ref, b2_ref, go_ref, to_ref, eps_ref, out_ref):
    h = h_ref[...]
    agg = a_ref[0, 0:N, :] + a_ref[1, 0:N, :]
    z0 = (1.0 + eps_ref[0, 0]) * h + agg
    z1 = jnp.dot(z0, w1_ref[...], preferred_element_type=jnp.float32) + b1_ref[...]
    mu = jnp.mean(z1, axis=0, keepdims=True)
    var = jnp.mean((z1 - mu) ** 2, axis=0, keepdims=True)
    z1 = (z1 - mu) / jnp.sqrt(var + 1e-5) * g1_ref[...] + t1_ref[...]
    z1 = jnp.maximum(z1, 0.0)
    z2 = jnp.dot(z1, w2_ref[...], preferred_element_type=jnp.float32) + b2_ref[...]
    mu2 = jnp.mean(z2, axis=0, keepdims=True)
    var2 = jnp.mean((z2 - mu2) ** 2, axis=0, keepdims=True)
    z2 = (z2 - mu2) / jnp.sqrt(var2 + 1e-5) * go_ref[...] + to_ref[...]
    if relu_out:
        z2 = jnp.maximum(z2, 0.0)
    out_ref[...] = z2 + h


def _mlp_call(h, parts, w1, b1v, g1v, t1v, w2, b2v, gov, tov, eps_l, relu_out):
    body = functools.partial(_mlp_body, relu_out)
    vspec = pl.BlockSpec(memory_space=pltpu.VMEM)
    return pl.pallas_call(
        body,
        out_shape=jax.ShapeDtypeStruct((N, D), jnp.float32),
        in_specs=[vspec] * 10 + [pl.BlockSpec(memory_space=pltpu.SMEM)],
        out_specs=vspec,
    )(h, parts, w1, b1v, g1v, t1v, w2, b2v, gov, tov, eps_l)


def kernel(x, edge_index, edge_attr, batch, eps, W1, b1, g1, bt1, W2, b2, bond_emb, g_out, bt_out):
    src = edge_index[0]
    dst = edge_index[1]
    cidx = edge_attr[:, 0] * 25 + edge_attr[:, 1] * 5 + edge_attr[:, 2]
    srcp = jnp.pad(src, (0, EPAD - E)).reshape(NW, G, 1, EPG)
    cidxp = jnp.pad(cidx, (0, EPAD - E)).reshape(NW, G, 1, EPG)
    dstp = jnp.pad(dst, (0, EPAD - E), constant_values=N).reshape(NW, G, 1, EPG)
    idxp = jnp.concatenate([srcp, cidxp, dstp], axis=2).reshape(NW, G, 3 * EPG)
    ctab = (bond_emb[:, 0][:, :, None, None, :]
            + bond_emb[:, 1][:, None, :, None, :]
            + bond_emb[:, 2][:, None, None, :, :]).reshape(L, 125, D)
    ctab = jnp.pad(ctab, ((0, 0), (0, 3), (0, 0)))

    h = x
    for l in range(L):
        parts = _mp_call(h, idxp, ctab[l])
        h = _mlp_call(h, parts,
                      W1[l], b1[l][None], g1[l][None], bt1[l][None],
                      W2[l], b2[l][None], g_out[l][None], bt_out[l][None],
                      eps[l].reshape(1, 1), relu_out=(l < L - 1))
    return h
